# baseline (speedup 1.0000x reference)
import functools

import jax
import jax.numpy as jnp
from jax import lax
from jax.experimental import pallas as pl
from jax.experimental.pallas import tpu as pltpu
from jax.experimental.pallas import tpu_sc as plsc

NS = 16
NC = 2
HALF = 128
BATCH = 16
NBUF = 4
NIDX = 8
GD = 2
SD = 2


def _sc_prop_kernel(nb, nacc, rows_per_tile):
  nchunk = rows_per_tile // BATCH
  mesh = plsc.VectorSubcoreMesh(core_axis_name="c", subcore_axis_name="s")

  @functools.partial(
      pl.kernel,
      mesh=mesh,
      out_type=jax.ShapeDtypeStruct((NC, nacc, HALF), jnp.float32),
      scratch_types=[
          pltpu.VMEM((NIDX, 3, BATCH), jnp.int32),
          pltpu.VMEM((NBUF, BATCH, HALF), jnp.float32),
          pltpu.VMEM((NBUF, BATCH, HALF), jnp.float32),
          pltpu.VMEM_SHARED((nacc, HALF), jnp.float32),
          pltpu.SemaphoreType.DMA((NIDX,)),
          pltpu.SemaphoreType.DMA((NBUF,)),
          pltpu.SemaphoreType.DMA((NBUF,)),
          pltpu.SemaphoreType.DMA((NBUF,)),
      ],
  )
  def sc_prop(xl, xr, embl, embr, idxp, out,
              idx_v, rows_v, scale_v, acc_sh, isem, gsem, g2sem, ssem):
    c = lax.axis_index("c")
    s = lax.axis_index("s")
    r_base = s * rows_per_tile

    def _start_idx(b, ki):
      pltpu.async_copy(idxp.at[s, b], idx_v.at[ki], isem.at[ki])

    def _wait_idx(b, ki):
      pltpu.make_async_copy(idxp.at[s, b], idx_v.at[ki], isem.at[ki]).wait()

    def _start_gather(kr, ki):
      @pl.when(c == 0)
      def _ga():
        pltpu.async_copy(xl.at[idx_v.at[ki, 0]], rows_v.at[kr], gsem.at[kr])
        pltpu.async_copy(embl.at[idx_v.at[ki, 2]], scale_v.at[kr],
                         g2sem.at[kr])

      @pl.when(c == 1)
      def _gb():
        pltpu.async_copy(xr.at[idx_v.at[ki, 0]], rows_v.at[kr], gsem.at[kr])
        pltpu.async_copy(embr.at[idx_v.at[ki, 2]], scale_v.at[kr],
                         g2sem.at[kr])

    def _wait_gather(kr, ki):
      @pl.when(c == 0)
      def _wa():
        pltpu.make_async_copy(xl.at[idx_v.at[ki, 0]], rows_v.at[kr],
                              gsem.at[kr]).wait()
        pltpu.make_async_copy(embl.at[idx_v.at[ki, 2]], scale_v.at[kr],
                              g2sem.at[kr]).wait()

      @pl.when(c == 1)
      def _wb():
        pltpu.make_async_copy(xr.at[idx_v.at[ki, 0]], rows_v.at[kr],
                              gsem.at[kr]).wait()
        pltpu.make_async_copy(embr.at[idx_v.at[ki, 2]], scale_v.at[kr],
                              g2sem.at[kr]).wait()

    def _start_scatter(kr, ki):
      pltpu.async_copy(rows_v.at[kr], acc_sh.at[idx_v.at[ki, 1]],
                       ssem.at[kr], add=True)

    def _wait_scatter(kr, ki):
      pltpu.make_async_copy(rows_v.at[kr], acc_sh.at[idx_v.at[ki, 1]],
                            ssem.at[kr]).wait()

    def _compute(kr):
      rv = rows_v.at[kr]
      sv = scale_v.at[kr]
      for i in range(BATCH):
        for j in range(HALF // 16):
          sl = pl.ds(j * 16, 16)
          rv[i, sl] = rv[i, sl] * sv[i, sl]

    zero16 = jnp.zeros((16,), jnp.float32)

    def _zrow(i, carry):
      for j in range(HALF // 16):
        scale_v[0, i, pl.ds(j * 16, 16)] = zero16
      return carry

    lax.fori_loop(0, BATCH, _zrow, 0)

    def _zacc(k, carry):
      pltpu.sync_copy(scale_v.at[0],
                      acc_sh.at[pl.ds(r_base + k * BATCH, BATCH)])
      return carry

    lax.fori_loop(0, nchunk, _zacc, 0)
    plsc.subcore_barrier()

    for bb in range(GD + 2):
      _start_idx(bb, bb)
    for bb in range(GD):
      _wait_idx(bb, bb)
      _start_gather(bb % NBUF, bb)

    def _body(b, carry):
      kr = lax.bitwise_and(b, NBUF - 1)
      ki = lax.bitwise_and(b, NIDX - 1)

      @pl.when(b >= SD)
      def _ws():
        _wait_scatter(lax.bitwise_and(b - SD, NBUF - 1),
                      lax.bitwise_and(b - SD, NIDX - 1))

      @pl.when(b + GD + 2 < nb)
      def _si():
        _start_idx(b + GD + 2, lax.bitwise_and(b + GD + 2, NIDX - 1))

      @pl.when(b + GD < nb)
      def _sg():
        _wait_idx(b + GD, lax.bitwise_and(b + GD, NIDX - 1))
        _start_gather(lax.bitwise_and(b + GD, NBUF - 1),
                      lax.bitwise_and(b + GD, NIDX - 1))

      _wait_gather(kr, ki)
      _compute(kr)
      _start_scatter(kr, ki)
      return carry

    lax.fori_loop(0, nb, _body, 0)
    for bb in range(SD):
      _wait_scatter((nb - SD + bb) % NBUF, (nb - SD + bb) % NIDX)
    plsc.subcore_barrier()

    def _emit(k, carry):
      r0 = r_base + k * BATCH
      pltpu.sync_copy(acc_sh.at[pl.ds(r0, BATCH)], rows_v.at[0])
      pltpu.sync_copy(rows_v.at[0], out.at[c, pl.ds(r0, BATCH)])
      return carry

    lax.fori_loop(0, nchunk, _emit, 0)

  return sc_prop


def _tc_body(x_b, pl_b, pr_b, wlT, wr1, wr2, bias, o_b):
  o_b[...] = (
      jnp.dot(x_b[...], wlT[...], preferred_element_type=jnp.float32)
      + jnp.dot(pl_b[...], wr1[...], preferred_element_type=jnp.float32)
      + jnp.dot(pr_b[...], wr2[...], preferred_element_type=jnp.float32)
      + bias[...]
  )


def kernel(x, edge_index, edge_weight, Wl, bl, Wr, br, emb):
  n, d = x.shape
  e = edge_index.shape[1]
  assert d == NC * HALF

  nb = NIDX * (-(-e // (NS * BATCH * NIDX)))
  epad = NS * nb * BATCH
  nacc = NS * BATCH * (-(-(n + 1) // (NS * BATCH)))
  rows_per_tile = nacc // NS

  src = jnp.pad(edge_index[0], (0, epad - e)).reshape(NS, nb, 1, BATCH)
  dst = jnp.pad(edge_index[1], (0, epad - e),
                constant_values=n).reshape(NS, nb, 1, BATCH)
  w = jnp.pad(edge_weight, (0, epad - e)).reshape(NS, nb, 1, BATCH)
  idxp = jnp.concatenate([src, dst, w], axis=2)

  xl = x[:, :HALF]
  xr = x[:, HALF:]
  embl = emb[:, :HALF]
  embr = emb[:, HALF:]

  prop2 = _sc_prop_kernel(nb, nacc, rows_per_tile)(
      xl, xr, embl, embr, idxp)
  prop_l = prop2[0, :n]
  prop_r = prop2[1, :n]

  wlT = Wl.T
  wrT = Wr.T
  bias = (bl + br).reshape(1, d)

  rb = 200 if n % 200 == 0 else 8 * (-(-n // 8))
  grid = n // rb if n % rb == 0 else 1
  if grid == 1:
    rb = n

  out = pl.pallas_call(
      _tc_body,
      grid=(grid,),
      in_specs=[
          pl.BlockSpec((rb, d), lambda i: (i, 0)),
          pl.BlockSpec((rb, HALF), lambda i: (i, 0)),
          pl.BlockSpec((rb, HALF), lambda i: (i, 0)),
          pl.BlockSpec((d, d), lambda i: (0, 0)),
          pl.BlockSpec((HALF, d), lambda i: (0, 0)),
          pl.BlockSpec((HALF, d), lambda i: (0, 0)),
          pl.BlockSpec((1, d), lambda i: (0, 0)),
      ],
      out_specs=pl.BlockSpec((rb, d), lambda i: (i, 0)),
      out_shape=jax.ShapeDtypeStruct((n, d), jnp.float32),
  )(x, prop_l, prop_r, wlT, wrT[:HALF], wrT[HALF:], bias)
  return out


# speedup vs baseline: 1.7458x; 1.7458x over previous
import functools

import jax
import jax.numpy as jnp
from jax import lax
from jax.experimental import pallas as pl
from jax.experimental.pallas import tpu as pltpu
from jax.experimental.pallas import tpu_sc as plsc

NS = 16
NC = 2
BATCH = 64
HALF = 128
NBUF = 4
NIDX = 8
GD = 2


def _sc_prop_kernel(nb, nacc, rows_per_tile):
  nzero = rows_per_tile // BATCH
  mesh = plsc.VectorSubcoreMesh(core_axis_name="c", subcore_axis_name="s")

  @functools.partial(
      pl.kernel,
      mesh=mesh,
      out_type=jax.ShapeDtypeStruct((NC, nacc, HALF), jnp.float32),
      scratch_types=[
          pltpu.VMEM((NIDX, 3, BATCH), jnp.int32),
          pltpu.VMEM((16, HALF), jnp.float32),
          pltpu.VMEM((NBUF, BATCH, HALF), jnp.float32),
          pltpu.VMEM_SHARED((nacc, HALF), jnp.float32),
          pltpu.SemaphoreType.DMA((NIDX,)),
          pltpu.SemaphoreType.DMA((NBUF,)),
          pltpu.SemaphoreType.DMA((NBUF,)),
      ],
  )
  def sc_prop(xl, xr, embl, embr, idxp, out,
              idx_v, emb_v, rows_v, acc_sh, isem, gsem, ssem):
    c = lax.axis_index("c")
    s = lax.axis_index("s")

    @pl.when(c == 0)
    def _stage_embl():
      pltpu.sync_copy(embl, emb_v.at[pl.ds(0, 10)])

    @pl.when(c == 1)
    def _stage_embr():
      pltpu.sync_copy(embr, emb_v.at[pl.ds(0, 10)])

    zero16 = jnp.zeros((16,), jnp.float32)

    def _zrow(i, carry):
      for j in range(HALF // 16):
        rows_v[0, i, pl.ds(j * 16, 16)] = zero16
      return carry

    lax.fori_loop(0, BATCH, _zrow, 0)

    def _zacc(k, carry):
      pltpu.sync_copy(rows_v.at[0],
                      acc_sh.at[pl.ds(s * rows_per_tile + k * BATCH, BATCH)])
      return carry

    lax.fori_loop(0, nzero, _zacc, 0)
    plsc.subcore_barrier()

    def _start_idx(b, ki):
      pltpu.async_copy(idxp.at[s, b], idx_v.at[ki], isem.at[ki])

    def _wait_idx(b, ki):
      pltpu.make_async_copy(idxp.at[s, b], idx_v.at[ki], isem.at[ki]).wait()

    def _start_gather(b, kr, ki):
      @pl.when(c == 0)
      def _gl():
        pltpu.async_copy(xl.at[idx_v.at[ki, 0]], rows_v.at[kr], gsem.at[kr])

      @pl.when(c == 1)
      def _gr():
        pltpu.async_copy(xr.at[idx_v.at[ki, 0]], rows_v.at[kr], gsem.at[kr])

    def _wait_gather(kr, ki):
      @pl.when(c == 0)
      def _wl():
        pltpu.make_async_copy(xl.at[idx_v.at[ki, 0]], rows_v.at[kr],
                              gsem.at[kr]).wait()

      @pl.when(c == 1)
      def _wr():
        pltpu.make_async_copy(xr.at[idx_v.at[ki, 0]], rows_v.at[kr],
                              gsem.at[kr]).wait()

    def _start_scatter(kr, ki):
      pltpu.async_copy(rows_v.at[kr], acc_sh.at[idx_v.at[ki, 1]],
                       ssem.at[kr], add=True)

    def _wait_scatter(kr, ki):
      pltpu.make_async_copy(rows_v.at[kr], acc_sh.at[idx_v.at[ki, 1]],
                            ssem.at[kr]).wait()

    def _compute(kr, ki):
      rv = rows_v.at[kr]
      wrow = idx_v.at[ki, 2]

      @plsc.parallel_loop(0, BATCH, 16, unroll=2)
      def _group(g):
        w16 = wrow[pl.ds(g, 16)]
        for i in range(16):
          w = w16[i]
          for j in range(HALF // 16):
            sl = pl.ds(j * 16, 16)
            rv[g + i, sl] = rv[g + i, sl] * emb_v[w, sl]

    for bb in range(GD + 2):
      _start_idx(bb, bb)
    for bb in range(GD):
      _wait_idx(bb, bb)
      _start_gather(bb, bb % NBUF, bb)

    def _body(b, carry):
      kr = lax.bitwise_and(b, NBUF - 1)
      ki = lax.bitwise_and(b, NIDX - 1)

      @pl.when(b >= 2)
      def _ws():
        _wait_scatter(lax.bitwise_and(b - 2, NBUF - 1),
                      lax.bitwise_and(b - 2, NIDX - 1))

      @pl.when(b + GD + 2 < nb)
      def _si():
        _start_idx(b + GD + 2, lax.bitwise_and(b + GD + 2, NIDX - 1))

      @pl.when(b + GD < nb)
      def _sg():
        _wait_idx(b + GD, lax.bitwise_and(b + GD, NIDX - 1))
        _start_gather(b + GD, lax.bitwise_and(b + GD, NBUF - 1),
                      lax.bitwise_and(b + GD, NIDX - 1))

      _wait_gather(kr, ki)
      _compute(kr, ki)
      _start_scatter(kr, ki)
      return carry

    lax.fori_loop(0, nb, _body, 0)
    _wait_scatter((nb - 2) % NBUF, (nb - 2) % NIDX)
    _wait_scatter((nb - 1) % NBUF, (nb - 1) % NIDX)
    plsc.subcore_barrier()

    def _emit(k, carry):
      r0 = s * rows_per_tile + k * BATCH
      pltpu.sync_copy(acc_sh.at[pl.ds(r0, BATCH)], rows_v.at[0])
      pltpu.sync_copy(rows_v.at[0], out.at[c, pl.ds(r0, BATCH)])
      return carry

    lax.fori_loop(0, nzero, _emit, 0)

  return sc_prop


def _tc_body(x_b, pl_b, pr_b, wlT, wr1, wr2, bias, o_b):
  o_b[...] = (
      jnp.dot(x_b[...], wlT[...], preferred_element_type=jnp.float32)
      + jnp.dot(pl_b[...], wr1[...], preferred_element_type=jnp.float32)
      + jnp.dot(pr_b[...], wr2[...], preferred_element_type=jnp.float32)
      + bias[...]
  )


def kernel(x, edge_index, edge_weight, Wl, bl, Wr, br, emb):
  n, d = x.shape
  e = edge_index.shape[1]
  assert d == 2 * HALF

  nb = NIDX * (-(-e // (NS * BATCH * NIDX)))
  epad = NS * nb * BATCH
  nacc = NS * BATCH * (-(-(n + 1) // (NS * BATCH)))
  rows_per_tile = nacc // NS

  src = jnp.pad(edge_index[0], (0, epad - e)).reshape(NS, nb, 1, BATCH)
  dst = jnp.pad(edge_index[1], (0, epad - e),
                constant_values=n).reshape(NS, nb, 1, BATCH)
  w = jnp.pad(edge_weight, (0, epad - e)).reshape(NS, nb, 1, BATCH)
  idxp = jnp.concatenate([src, dst, w], axis=2)

  xl = x[:, :HALF]
  xr = x[:, HALF:]
  embl = emb[:, :HALF]
  embr = emb[:, HALF:]

  prop2 = _sc_prop_kernel(nb, nacc, rows_per_tile)(
      xl, xr, embl, embr, idxp)
  prop_l = prop2[0, :n]
  prop_r = prop2[1, :n]

  wlT = Wl.T
  wrT = Wr.T
  bias = (bl + br).reshape(1, d)

  rb = 200 if n % 200 == 0 else 8 * (-(-n // 8))
  grid = n // rb if n % rb == 0 else 1
  if grid == 1:
    rb = n

  out = pl.pallas_call(
      _tc_body,
      grid=(grid,),
      in_specs=[
          pl.BlockSpec((rb, d), lambda i: (i, 0)),
          pl.BlockSpec((rb, HALF), lambda i: (i, 0)),
          pl.BlockSpec((rb, HALF), lambda i: (i, 0)),
          pl.BlockSpec((d, d), lambda i: (0, 0)),
          pl.BlockSpec((HALF, d), lambda i: (0, 0)),
          pl.BlockSpec((HALF, d), lambda i: (0, 0)),
          pl.BlockSpec((1, d), lambda i: (0, 0)),
      ],
      out_specs=pl.BlockSpec((rb, d), lambda i: (i, 0)),
      out_shape=jax.ShapeDtypeStruct((n, d), jnp.float32),
  )(x, prop_l, prop_r, wlT, wrT[:HALF], wrT[HALF:], bias)
  return out


# speedup vs baseline: 2.0103x; 1.1515x over previous
import functools

import jax
import jax.numpy as jnp
from jax import lax
from jax.experimental import pallas as pl
from jax.experimental.pallas import tpu as pltpu
from jax.experimental.pallas import tpu_sc as plsc

NS = 16
NC = 2
BATCH = 64
HALF = 128
NBUF = 4
NIDX = 8
GD = 2


def _sc_prop_kernel(nb, nacc, rows_per_tile):
  nzero = rows_per_tile // BATCH
  mesh = plsc.VectorSubcoreMesh(core_axis_name="c", subcore_axis_name="s")

  @functools.partial(
      pl.kernel,
      mesh=mesh,
      out_type=jax.ShapeDtypeStruct((NC, nacc, HALF), jnp.float32),
      scratch_types=[
          pltpu.VMEM((NIDX, 3, BATCH), jnp.int32),
          pltpu.VMEM((16, HALF), jnp.float32),
          pltpu.VMEM((NBUF, BATCH, HALF), jnp.float32),
          pltpu.VMEM_SHARED((nacc, HALF), jnp.float32),
          pltpu.SemaphoreType.DMA((NIDX,)),
          pltpu.SemaphoreType.DMA((NBUF,)),
          pltpu.SemaphoreType.DMA((NBUF,)),
      ],
  )
  def sc_prop(xl, xr, embl, embr, idxp, out,
              idx_v, emb_v, rows_v, acc_sh, isem, gsem, ssem):
    c = lax.axis_index("c")
    s = lax.axis_index("s")

    @pl.when(c == 0)
    def _stage_embl():
      pltpu.sync_copy(embl, emb_v.at[pl.ds(0, 10)])

    @pl.when(c == 1)
    def _stage_embr():
      pltpu.sync_copy(embr, emb_v.at[pl.ds(0, 10)])

    zero16 = jnp.zeros((16,), jnp.float32)

    def _zrow(i, carry):
      for j in range(HALF // 16):
        rows_v[0, i, pl.ds(j * 16, 16)] = zero16
      return carry

    lax.fori_loop(0, BATCH, _zrow, 0)

    def _zacc(k, carry):
      pltpu.sync_copy(rows_v.at[0],
                      acc_sh.at[pl.ds(s * rows_per_tile + k * BATCH, BATCH)])
      return carry

    lax.fori_loop(0, nzero, _zacc, 0)
    plsc.subcore_barrier()

    def _start_idx(b, ki):
      pltpu.async_copy(idxp.at[s, b], idx_v.at[ki], isem.at[ki])

    def _wait_idx(b, ki):
      pltpu.make_async_copy(idxp.at[s, b], idx_v.at[ki], isem.at[ki]).wait()

    def _start_gather(b, kr, ki):
      @pl.when(c == 0)
      def _gl():
        pltpu.async_copy(xl.at[idx_v.at[ki, 0]], rows_v.at[kr], gsem.at[kr])

      @pl.when(c == 1)
      def _gr():
        pltpu.async_copy(xr.at[idx_v.at[ki, 0]], rows_v.at[kr], gsem.at[kr])

    def _wait_gather(kr, ki):
      @pl.when(c == 0)
      def _wl():
        pltpu.make_async_copy(xl.at[idx_v.at[ki, 0]], rows_v.at[kr],
                              gsem.at[kr]).wait()

      @pl.when(c == 1)
      def _wr():
        pltpu.make_async_copy(xr.at[idx_v.at[ki, 0]], rows_v.at[kr],
                              gsem.at[kr]).wait()

    def _start_scatter(kr, ki):
      pltpu.async_copy(rows_v.at[kr], acc_sh.at[idx_v.at[ki, 1]],
                       ssem.at[kr], add=True)

    def _wait_scatter(kr, ki):
      pltpu.make_async_copy(rows_v.at[kr], acc_sh.at[idx_v.at[ki, 1]],
                            ssem.at[kr]).wait()

    def _compute(kr, ki):
      rv = rows_v.at[kr]
      wrow = idx_v.at[ki, 2]

      @plsc.parallel_loop(0, BATCH, 16, unroll=4)
      def _group(g):
        w16 = wrow[pl.ds(g, 16)]
        for i in range(16):
          w = w16[i]
          for j in range(HALF // 16):
            sl = pl.ds(j * 16, 16)
            rv[g + i, sl] = rv[g + i, sl] * emb_v[w, sl]

    for bb in range(GD + 2):
      _start_idx(bb, bb)
    for bb in range(GD):
      _wait_idx(bb, bb)
      _start_gather(bb, bb % NBUF, bb)

    def _body(b, carry):
      kr = lax.bitwise_and(b, NBUF - 1)
      ki = lax.bitwise_and(b, NIDX - 1)

      @pl.when(b >= 2)
      def _ws():
        _wait_scatter(lax.bitwise_and(b - 2, NBUF - 1),
                      lax.bitwise_and(b - 2, NIDX - 1))

      @pl.when(b + GD + 2 < nb)
      def _si():
        _start_idx(b + GD + 2, lax.bitwise_and(b + GD + 2, NIDX - 1))

      @pl.when(b + GD < nb)
      def _sg():
        _wait_idx(b + GD, lax.bitwise_and(b + GD, NIDX - 1))
        _start_gather(b + GD, lax.bitwise_and(b + GD, NBUF - 1),
                      lax.bitwise_and(b + GD, NIDX - 1))

      _wait_gather(kr, ki)
      _compute(kr, ki)
      _start_scatter(kr, ki)
      return carry

    lax.fori_loop(0, nb, _body, 0)
    _wait_scatter((nb - 2) % NBUF, (nb - 2) % NIDX)
    _wait_scatter((nb - 1) % NBUF, (nb - 1) % NIDX)
    plsc.subcore_barrier()

    def _emit(k, carry):
      r0 = s * rows_per_tile + k * BATCH
      pltpu.sync_copy(acc_sh.at[pl.ds(r0, BATCH)], rows_v.at[0])
      pltpu.sync_copy(rows_v.at[0], out.at[c, pl.ds(r0, BATCH)])
      return carry

    lax.fori_loop(0, nzero, _emit, 0)

  return sc_prop


def _tc_body(x_b, pl_b, pr_b, wlT, wr1, wr2, bias, o_b):
  o_b[...] = (
      jnp.dot(x_b[...], wlT[...], preferred_element_type=jnp.float32)
      + jnp.dot(pl_b[...], wr1[...], preferred_element_type=jnp.float32)
      + jnp.dot(pr_b[...], wr2[...], preferred_element_type=jnp.float32)
      + bias[...]
  )


def kernel(x, edge_index, edge_weight, Wl, bl, Wr, br, emb):
  n, d = x.shape
  e = edge_index.shape[1]
  assert d == 2 * HALF

  nb = NIDX * (-(-e // (NS * BATCH * NIDX)))
  epad = NS * nb * BATCH
  nacc = NS * BATCH * (-(-(n + 1) // (NS * BATCH)))
  rows_per_tile = nacc // NS

  src = jnp.pad(edge_index[0], (0, epad - e)).reshape(NS, nb, 1, BATCH)
  dst = jnp.pad(edge_index[1], (0, epad - e),
                constant_values=n).reshape(NS, nb, 1, BATCH)
  w = jnp.pad(edge_weight, (0, epad - e)).reshape(NS, nb, 1, BATCH)
  idxp = jnp.concatenate([src, dst, w], axis=2)

  xl = x[:, :HALF]
  xr = x[:, HALF:]
  embl = emb[:, :HALF]
  embr = emb[:, HALF:]

  prop2 = _sc_prop_kernel(nb, nacc, rows_per_tile)(
      xl, xr, embl, embr, idxp)
  prop_l = prop2[0, :n]
  prop_r = prop2[1, :n]

  wlT = Wl.T
  wrT = Wr.T
  bias = (bl + br).reshape(1, d)

  rb = 200 if n % 200 == 0 else 8 * (-(-n // 8))
  grid = n // rb if n % rb == 0 else 1
  if grid == 1:
    rb = n

  out = pl.pallas_call(
      _tc_body,
      grid=(grid,),
      in_specs=[
          pl.BlockSpec((rb, d), lambda i: (i, 0)),
          pl.BlockSpec((rb, HALF), lambda i: (i, 0)),
          pl.BlockSpec((rb, HALF), lambda i: (i, 0)),
          pl.BlockSpec((d, d), lambda i: (0, 0)),
          pl.BlockSpec((HALF, d), lambda i: (0, 0)),
          pl.BlockSpec((HALF, d), lambda i: (0, 0)),
          pl.BlockSpec((1, d), lambda i: (0, 0)),
      ],
      out_specs=pl.BlockSpec((rb, d), lambda i: (i, 0)),
      out_shape=jax.ShapeDtypeStruct((n, d), jnp.float32),
  )(x, prop_l, prop_r, wlT, wrT[:HALF], wrT[HALF:], bias)
  return out
